# baseline (speedup 1.0000x reference)
import functools
import math

import jax
import jax.numpy as jnp
from jax import lax
from jax.experimental import pallas as pl
from jax.experimental.pallas import tpu as pltpu
from jax.experimental.pallas import tpu_sc as plsc

_B = 16384
_D = 16
_PERIOD = 365.25
_NC, _NS = 2, 16
_NW = _NC * _NS
_BPW = _B // _NW
_CH = 128
_NCHUNK = _BPW // _CH

_mesh = plsc.VectorSubcoreMesh(core_axis_name="c", subcore_axis_name="s")


@functools.partial(
    pl.kernel,
    mesh=_mesh,
    out_type=[
        jax.ShapeDtypeStruct((_B, _D), jnp.float32),
        jax.ShapeDtypeStruct((_B, _D), jnp.float32),
    ],
    scratch_types=[
        pltpu.VMEM((_BPW,), jnp.int32),
        pltpu.VMEM((_BPW, _D), jnp.float32),
        pltpu.VMEM((_BPW, _D), jnp.float32),
        pltpu.SemaphoreType.DMA,
    ],
    compiler_params=pltpu.CompilerParams(use_tc_tiling_on_sc=False),
)
def _sc_gather(idx_hbm, a_hbm, b_hbm, outa_hbm, outb_hbm, idx_v, ra_v, rb_v, sem):
    wid = lax.axis_index("s") * _NC + lax.axis_index("c")
    base = wid * _BPW
    pltpu.sync_copy(idx_hbm.at[pl.ds(base, _BPW)], idx_v)
    copies = []
    for j in range(_NCHUNK):
        sl = pl.ds(j * _CH, _CH)
        copies.append(pltpu.async_copy(a_hbm.at[idx_v.at[sl]], ra_v.at[sl], sem))
        copies.append(pltpu.async_copy(b_hbm.at[idx_v.at[sl]], rb_v.at[sl], sem))
    for c in copies:
        c.wait()
    pltpu.sync_copy(ra_v, outa_hbm.at[pl.ds(base, _BPW)])
    pltpu.sync_copy(rb_v, outb_hbm.at[pl.ds(base, _BPW)])


_ROWS = 2048


def _tc_combine(t_ref, a_ref, b_ref, o_ref):
    t = t_ref[...]
    n = lax.broadcasted_iota(jnp.int32, (_ROWS, _D), 1).astype(jnp.float32) + 1.0
    x = (2.0 * math.pi / _PERIOD) * (t * n)
    acc = jnp.cos(x) * a_ref[...] + jnp.sin(x) * b_ref[...]
    o_ref[...] = jnp.sum(acc, axis=1, keepdims=True)


def kernel(t, id, a_table, b_table):
    idx = id.reshape(-1).astype(jnp.int32)
    rows_a, rows_b = _sc_gather(idx, a_table, b_table)
    out = pl.pallas_call(
        _tc_combine,
        grid=(_B // _ROWS,),
        in_specs=[
            pl.BlockSpec((_ROWS, 1), lambda i: (i, 0)),
            pl.BlockSpec((_ROWS, _D), lambda i: (i, 0)),
            pl.BlockSpec((_ROWS, _D), lambda i: (i, 0)),
        ],
        out_specs=pl.BlockSpec((_ROWS, 1), lambda i: (i, 0)),
        out_shape=jax.ShapeDtypeStruct((_B, 1), jnp.float32),
    )(t, rows_a, rows_b)
    return out


# speedup vs baseline: 6.2144x; 6.2144x over previous
import functools
import math

import jax
import jax.numpy as jnp
from jax import lax
from jax.experimental import pallas as pl
from jax.experimental.pallas import tpu as pltpu
from jax.experimental.pallas import tpu_sc as plsc

_B = 16384
_D = 16
_PERIOD = 365.25
_NC, _NS = 2, 16
_NW = _NC * _NS
_BPW = _B // _NW
_W = 8
_NPAIR = _BPW // (2 * _W)

_mesh = plsc.VectorSubcoreMesh(core_axis_name="c", subcore_axis_name="s")

_lane = lambda: lax.broadcasted_iota(jnp.int32, (16,), 0)


def _splat(v):
    return jax.lax.broadcast(v, (16,))


@functools.partial(
    pl.kernel,
    mesh=_mesh,
    out_type=[
        jax.ShapeDtypeStruct((_D, _B), jnp.float32),
        jax.ShapeDtypeStruct((_D, _B), jnp.float32),
    ],
    scratch_types=[
        pltpu.SMEM((_BPW,), jnp.int32),
        pltpu.VMEM((2 * _BPW,), jnp.int32),
        pltpu.VMEM((_W, _D, 128), jnp.float32),
        pltpu.VMEM((_W, _D, 128), jnp.float32),
        pltpu.VMEM((_W, _D, 128), jnp.float32),
        pltpu.VMEM((_W, _D, 128), jnp.float32),
        pltpu.VMEM((_D, _BPW), jnp.float32),
        pltpu.VMEM((_D, _BPW), jnp.float32),
        pltpu.SemaphoreType.DMA,
        pltpu.SemaphoreType.DMA,
        pltpu.SemaphoreType.DMA,
        pltpu.SemaphoreType.DMA,
    ],
    compiler_params=pltpu.CompilerParams(needs_layout_passes=False),
)
def _sc_gather(idx_hbm, a_hbm, b_hbm, oa, ob,
               idx_s, idx_v, ba0, bb0, ba1, bb1, rta, rtb,
               sa0, sb0, sa1, sb1):
    wid = lax.axis_index("s") * _NC + lax.axis_index("c")
    base = pl.multiple_of(wid * _BPW, _BPW)
    blk = pl.multiple_of((wid // 2) * (2 * _BPW), 2 * _BPW)
    pltpu.sync_copy(idx_hbm.at[pl.ds(blk, 2 * _BPW)], idx_v)
    half = (wid % 2) * _BPW
    for g in range(_BPW // 16):
        v = idx_v[pl.ds(half + g * 16, 16)]
        for l in range(16):
            idx_s[g * 16 + l] = v[l]

    def fire(w, ba, bb, sa, sb):
        for l in range(_W):
            i = idx_s[w * _W + l]
            c = pl.multiple_of((i // 128) * 128, 128)
            pltpu.async_copy(a_hbm.at[:, pl.ds(c, 128)], ba.at[l], sa)
            pltpu.async_copy(b_hbm.at[:, pl.ds(c, 128)], bb.at[l], sb)

    def drain(ba, bb, sa, sb):
        for l in range(_W):
            pltpu.make_async_copy(a_hbm.at[:, pl.ds(0, 128)], ba.at[l], sa).wait()
            pltpu.make_async_copy(b_hbm.at[:, pl.ds(0, 128)], bb.at[l], sb).wait()

    def extract(w, ba, bb):
        lane = _lane()
        for l in range(_W):
            r = w * _W + l
            col = _splat(idx_s[r] % 128)
            rs = _splat(r)
            slot = jnp.full((16,), l, jnp.int32)
            va = plsc.load_gather(ba, [slot, lane, col])
            plsc.store_scatter(rta, [lane, rs], va)
            vb = plsc.load_gather(bb, [slot, lane, col])
            plsc.store_scatter(rtb, [lane, rs], vb)

    fire(0, ba0, bb0, sa0, sb0)

    def body(k, _):
        w0 = 2 * k
        fire(w0 + 1, ba1, bb1, sa1, sb1)
        drain(ba0, bb0, sa0, sb0)
        extract(w0, ba0, bb0)

        @pl.when(k < _NPAIR - 1)
        def _():
            fire(w0 + 2, ba0, bb0, sa0, sb0)

        drain(ba1, bb1, sa1, sb1)
        extract(w0 + 1, ba1, bb1)
        return 0

    lax.fori_loop(0, _NPAIR, body, 0)
    pltpu.sync_copy(rta, oa.at[:, pl.ds(base, _BPW)])
    pltpu.sync_copy(rtb, ob.at[:, pl.ds(base, _BPW)])


_COLS = 2048


def _tc_combine(t_ref, a_ref, b_ref, o_ref):
    t = t_ref[...]
    n = lax.broadcasted_iota(jnp.int32, (_D, _COLS), 0).astype(jnp.float32) + 1.0
    x = (2.0 * math.pi / _PERIOD) * (n * t)
    acc = jnp.cos(x) * a_ref[...] + jnp.sin(x) * b_ref[...]
    o_ref[...] = jnp.sum(acc, axis=0, keepdims=True)


def kernel(t, id, a_table, b_table):
    idx = id.reshape(-1).astype(jnp.int32)
    rows_ta, rows_tb = _sc_gather(idx, a_table.T, b_table.T)
    out_t = pl.pallas_call(
        _tc_combine,
        grid=(_B // _COLS,),
        in_specs=[
            pl.BlockSpec((1, _COLS), lambda i: (0, i)),
            pl.BlockSpec((_D, _COLS), lambda i: (0, i)),
            pl.BlockSpec((_D, _COLS), lambda i: (0, i)),
        ],
        out_specs=pl.BlockSpec((1, _COLS), lambda i: (0, i)),
        out_shape=jax.ShapeDtypeStruct((1, _B), jnp.float32),
    )(t.reshape(1, _B), rows_ta, rows_tb)
    return out_t.reshape(_B, 1)


# speedup vs baseline: 7.9202x; 1.2745x over previous
import functools
import math

import jax
import jax.numpy as jnp
from jax import lax
from jax.experimental import pallas as pl
from jax.experimental.pallas import tpu as pltpu
from jax.experimental.pallas import tpu_sc as plsc

_B = 16384
_D = 16
_PERIOD = 365.25
_NC, _NS = 2, 16
_NW = _NC * _NS
_NTILE = 7813
_RANGE = 245
_WT = 8
_NWIN = 31
_MAXC = _NTILE - _WT
_FLT = 784

_mesh = plsc.VectorSubcoreMesh(core_axis_name="c", subcore_axis_name="s")
_lane = lambda: lax.broadcasted_iota(jnp.int32, (16,), 0)


def _splat(v):
    return jax.lax.broadcast(v, (16,))


def _take(v, i):
    return lax.gather(
        v, i.reshape(16, 1),
        lax.GatherDimensionNumbers(
            offset_dims=(), collapsed_slice_dims=(0,), start_index_map=(0,)),
        slice_sizes=(1,),
        mode=lax.GatherScatterMode.PROMISE_IN_BOUNDS)


@functools.partial(
    pl.kernel,
    mesh=_mesh,
    out_type=[
        jax.ShapeDtypeStruct((_B * _D,), jnp.float32),
        jax.ShapeDtypeStruct((_B * _D,), jnp.float32),
    ],
    scratch_types=[
        pltpu.VMEM((_B,), jnp.int32),
        pltpu.VMEM((_FLT + 16,), jnp.int32),
        pltpu.VMEM((_FLT + 16,), jnp.int32),
        pltpu.VMEM((_D, _WT * 128), jnp.float32),
        pltpu.VMEM((_D, _WT * 128), jnp.float32),
        pltpu.VMEM((_D, _WT * 128), jnp.float32),
        pltpu.VMEM((_D, _WT * 128), jnp.float32),
        pltpu.VMEM(((_FLT + 16) * _D,), jnp.float32),
        pltpu.VMEM(((_FLT + 16) * _D,), jnp.float32),
        pltpu.VMEM((256,), jnp.float32),
        pltpu.SMEM((_FLT + 16,), jnp.int32),
        pltpu.SemaphoreType.DMA,
        pltpu.SemaphoreType.DMA,
        pltpu.SemaphoreType.DMA,
        pltpu.SemaphoreType.DMA,
    ],
    compiler_params=pltpu.CompilerParams(needs_layout_passes=False),
)
def _sc_gather(idx_hbm, a_hbm, b_hbm, oa, ob,
               ids_v, flt_v, wl_v, ba0, bb0, ba1, bb1, ca_v, cb_v, dummy_v,
               bs_s, sa0, sb0, sa1, sb1):
    wid = lax.axis_index("s") * _NC + lax.axis_index("c")
    lo = wid * _RANGE
    lane = _lane()

    def filt(g_, off):
        v = ids_v[pl.ds(g_ * 16, 16)]
        tg = v // 128
        tl = tg - lo
        m = (tl >= 0) & (tl < _RANGE)
        packed = (tl << 21) | ((v % 128) << 14) | (g_ * 16 + lane)
        plsc.store_compressed(
            flt_v.at[pl.ds(jnp.minimum(off, _FLT), 16)], packed, mask=m)
        cnt = plsc.all_reduce_population_count(m)
        return off + cnt[0]

    def fire(w, ba, bb, sa, sb):
        cs = pl.multiple_of(jnp.minimum(lo + w * _WT, _MAXC) * 128, 128)
        pltpu.async_copy(a_hbm.at[:, pl.ds(cs, _WT * 128)], ba, sa)
        pltpu.async_copy(b_hbm.at[:, pl.ds(cs, _WT * 128)], bb, sb)

    def drain(ba, bb, sa, sb):
        pltpu.make_async_copy(a_hbm.at[:, pl.ds(0, _WT * 128)], ba, sa).wait()
        pltpu.make_async_copy(b_hbm.at[:, pl.ds(0, _WT * 128)], bb, sb).wait()

    def extract(w, ba, bb, n, m0):
        cs_t = jnp.minimum(lo + w * _WT, _MAXC)
        sbase = cs_t - lo

        def scan(g_, off):
            p = flt_v[pl.ds(g_ * 16, 16)]
            tl = p >> 21
            m = (tl >= sbase) & (tl < sbase + _WT) & (g_ * 16 + lane < n)
            plsc.store_compressed(wl_v.at[pl.ds(off, 16)], p, mask=m)
            cnt = plsc.all_reduce_population_count(m)
            return off + cnt[0]

        nw = lax.fori_loop(0, (n + 15) // 16, scan, 0)

        def one(k, m_):
            ch = wl_v[pl.ds((k // 16) * 16, 16)]
            e = _take(ch, _splat(k % 16))
            col = ((e >> 21) - sbase) * 128 + ((e >> 14) & 127)
            bpos = e & 16383
            va = plsc.load_gather(ba, [lane, col])
            vb = plsc.load_gather(bb, [lane, col])
            dst = _splat(m_ * 16) + lane
            plsc.store_scatter(ca_v, [dst], va)
            plsc.store_scatter(cb_v, [dst], vb)
            bs_s[m_] = bpos[0] * 16
            return m_ + 1

        return lax.fori_loop(0, nw, one, m0)

    fire(0, ba0, bb0, sa0, sb0)
    fire(1, ba1, bb1, sa1, sb1)

    pltpu.sync_copy(idx_hbm, ids_v)
    n = jnp.minimum(lax.fori_loop(0, _B // 16, filt, 0), _FLT)

    def wbody(k, m_):
        w0 = 2 * k
        drain(ba0, bb0, sa0, sb0)
        m_ = extract(w0, ba0, bb0, n, m_)

        @pl.when(w0 + 2 < _NWIN)
        def _():
            fire(w0 + 2, ba0, bb0, sa0, sb0)

        drain(ba1, bb1, sa1, sb1)
        m_ = extract(w0 + 1, ba1, bb1, n, m_)

        @pl.when(w0 + 3 < _NWIN)
        def _():
            fire(w0 + 3, ba1, bb1, sa1, sb1)

        return m_

    m = lax.fori_loop(0, _NWIN // 2, wbody, 0)
    drain(ba0, bb0, sa0, sb0)
    m = extract(_NWIN - 1, ba0, bb0, n, m)

    m16 = ((m + 15) // 16) * 16
    row0a = ca_v[pl.ds(0, 16)]
    row0b = cb_v[pl.ds(0, 16)]
    off0 = bs_s[0]

    def pad(p, _):
        dst = _splat(p * 16) + lane
        plsc.store_scatter(ca_v, [dst], row0a)
        plsc.store_scatter(cb_v, [dst], row0b)
        bs_s[p] = off0
        return 0

    lax.fori_loop(m, m16, pad, 0)

    def out_win(q, _):
        for l in range(16):
            j = q * 16 + l
            off = pl.multiple_of(bs_s[j], 16)
            pltpu.async_copy(ca_v.at[pl.ds(j * 16, 16)], oa.at[pl.ds(off, 16)], sa0)
            pltpu.async_copy(cb_v.at[pl.ds(j * 16, 16)], ob.at[pl.ds(off, 16)], sb0)

        @pl.when(q > 0)
        def _():
            pltpu.make_async_copy(oa.at[pl.ds(0, 256)], dummy_v, sa0).wait()
            pltpu.make_async_copy(ob.at[pl.ds(0, 256)], dummy_v, sb0).wait()

        return 0

    lax.fori_loop(0, m16 // 16, out_win, 0)

    @pl.when(m16 > 0)
    def _():
        pltpu.make_async_copy(oa.at[pl.ds(0, 256)], dummy_v, sa0).wait()
        pltpu.make_async_copy(ob.at[pl.ds(0, 256)], dummy_v, sb0).wait()


_ROWS = _B * _D // 128
_BR = 256


def _tc_combine(t_ref, a_ref, b_ref, o_ref):
    li = lax.broadcasted_iota(jnp.int32, (128, 8), 0)
    ji = lax.broadcasted_iota(jnp.int32, (128, 8), 1)
    s = (li // 16 == ji).astype(jnp.float32)
    lit = lax.broadcasted_iota(jnp.int32, (8, 128), 1)
    jit = lax.broadcasted_iota(jnp.int32, (8, 128), 0)
    st = (lit // 16 == jit).astype(jnp.float32)
    t = jnp.dot(t_ref[...], st, preferred_element_type=jnp.float32)
    n = (lax.broadcasted_iota(jnp.int32, (_BR, 128), 1) % 16 + 1).astype(
        jnp.float32)
    x = (2.0 * math.pi / _PERIOD) * (n * t)
    acc = jnp.cos(x) * a_ref[...] + jnp.sin(x) * b_ref[...]
    o_ref[...] = jnp.dot(acc, s, preferred_element_type=jnp.float32)


def kernel(t, id, a_table, b_table):
    idx = id.reshape(-1).astype(jnp.int32)
    ra, rb = _sc_gather(idx, a_table.T, b_table.T)
    out2 = pl.pallas_call(
        _tc_combine,
        grid=(_ROWS // _BR,),
        in_specs=[
            pl.BlockSpec((_BR, 8), lambda i: (i, 0)),
            pl.BlockSpec((_BR, 128), lambda i: (i, 0)),
            pl.BlockSpec((_BR, 128), lambda i: (i, 0)),
        ],
        out_specs=pl.BlockSpec((_BR, 8), lambda i: (i, 0)),
        out_shape=jax.ShapeDtypeStruct((_ROWS, 8), jnp.float32),
    )(t.reshape(_ROWS, 8), ra.reshape(_ROWS, 128), rb.reshape(_ROWS, 128))
    return out2.reshape(_B, 1)


# speedup vs baseline: 8.3914x; 1.0595x over previous
import functools
import math

import jax
import jax.numpy as jnp
from jax import lax
from jax.experimental import pallas as pl
from jax.experimental.pallas import tpu as pltpu
from jax.experimental.pallas import tpu_sc as plsc

_B = 16384
_D = 16
_PERIOD = 365.25
_NC, _NS = 2, 16
_NW = _NC * _NS
_NTILE = 7813
_RANGE = 245
_WT = 8
_NWIN = 31
_MAXC = _NTILE - _WT
_FLT = 784

_mesh = plsc.VectorSubcoreMesh(core_axis_name="c", subcore_axis_name="s")
_lane = lambda: lax.broadcasted_iota(jnp.int32, (16,), 0)


def _splat(v):
    return jax.lax.broadcast(v, (16,))


def _take(v, i):
    return lax.gather(
        v, i.reshape(16, 1),
        lax.GatherDimensionNumbers(
            offset_dims=(), collapsed_slice_dims=(0,), start_index_map=(0,)),
        slice_sizes=(1,),
        mode=lax.GatherScatterMode.PROMISE_IN_BOUNDS)


@functools.partial(
    pl.kernel,
    mesh=_mesh,
    out_type=[
        jax.ShapeDtypeStruct((_B * _D,), jnp.float32),
        jax.ShapeDtypeStruct((_B * _D,), jnp.float32),
    ],
    scratch_types=[
        pltpu.VMEM((_B,), jnp.int32),
        pltpu.VMEM((_FLT + 16,), jnp.int32),
        pltpu.VMEM((_FLT + 16,), jnp.int32),
        pltpu.VMEM((_D, _WT * 128), jnp.float32),
        pltpu.VMEM((_D, _WT * 128), jnp.float32),
        pltpu.VMEM((_D, _WT * 128), jnp.float32),
        pltpu.VMEM((_D, _WT * 128), jnp.float32),
        pltpu.VMEM(((_FLT + 16) * _D,), jnp.float32),
        pltpu.VMEM(((_FLT + 16) * _D,), jnp.float32),
        pltpu.VMEM((256,), jnp.float32),
        pltpu.SemaphoreType.DMA,
        pltpu.SemaphoreType.DMA,
        pltpu.SemaphoreType.DMA,
        pltpu.SemaphoreType.DMA,
        pltpu.SemaphoreType.DMA,
        pltpu.SemaphoreType.DMA,
    ],
    compiler_params=pltpu.CompilerParams(needs_layout_passes=False),
)
def _sc_gather(idx_hbm, a_hbm, b_hbm, oa, ob,
               ids_v, flt_v, wl_v, ba0, bb0, ba1, bb1, ca_v, cb_v, dummy_v,
               sa0, sb0, sa1, sb1, soa, sob):
    wid = lax.axis_index("s") * _NC + lax.axis_index("c")
    lo = wid * _RANGE
    lane = _lane()

    def filt(g_, off):
        v = ids_v[pl.ds(g_ * 16, 16)]
        roff = v - lo * 128
        m = (roff >= 0) & (roff < _RANGE * 128)
        packed = (roff << 14) | (g_ * 16 + lane)
        plsc.store_compressed(
            flt_v.at[pl.ds(jnp.minimum(off, _FLT), 16)], packed, mask=m)
        cnt = plsc.all_reduce_population_count(m)
        return off + cnt[0]

    def fire(w, ba, bb, sa, sb):
        cs = pl.multiple_of(jnp.minimum(lo + w * _WT, _MAXC) * 128, 128)
        pltpu.async_copy(a_hbm.at[:, pl.ds(cs, _WT * 128)], ba, sa)
        pltpu.async_copy(b_hbm.at[:, pl.ds(cs, _WT * 128)], bb, sb)

    def drain(ba, bb, sa, sb):
        pltpu.make_async_copy(a_hbm.at[:, pl.ds(0, _WT * 128)], ba, sa).wait()
        pltpu.make_async_copy(b_hbm.at[:, pl.ds(0, _WT * 128)], bb, sb).wait()

    def extract(w, ba, bb, n, m0):
        cs_t = jnp.minimum(lo + w * _WT, _MAXC)
        sbase = cs_t - lo

        def scan(g_, off):
            p = flt_v[pl.ds(g_ * 16, 16)]
            ro = p >> 14
            m = (ro >= sbase * 128) & (ro < (sbase + _WT) * 128) & (
                g_ * 16 + lane < n)
            plsc.store_compressed(wl_v.at[pl.ds(off, 16)], p, mask=m)
            cnt = plsc.all_reduce_population_count(m)
            return off + cnt[0]

        nw = lax.fori_loop(0, (n + 15) // 16, scan, 0)

        def one(k, m_):
            ch = wl_v[pl.ds((k // 16) * 16, 16)]
            e = _take(ch, _splat(k % 16))
            col = (e >> 14) - sbase * 128
            bpos = e & 16383
            va = plsc.load_gather(ba, [lane, col])
            vb = plsc.load_gather(bb, [lane, col])
            slot = m_ * 16
            dst = _splat(slot) + lane
            plsc.store_scatter(ca_v, [dst], va)
            plsc.store_scatter(cb_v, [dst], vb)
            off = pl.multiple_of(bpos[0] * 16, 16)
            pltpu.async_copy(ca_v.at[pl.ds(slot, 16)], oa.at[pl.ds(off, 16)], soa)
            pltpu.async_copy(cb_v.at[pl.ds(slot, 16)], ob.at[pl.ds(off, 16)], sob)
            return m_ + 1

        return lax.fori_loop(0, nw, one, m0)

    fire(0, ba0, bb0, sa0, sb0)
    fire(1, ba1, bb1, sa1, sb1)

    pltpu.sync_copy(idx_hbm, ids_v)
    n = jnp.minimum(lax.fori_loop(0, _B // 16, filt, 0), _FLT)

    def wbody(k, m_):
        w0 = 2 * k
        drain(ba0, bb0, sa0, sb0)
        m_ = extract(w0, ba0, bb0, n, m_)

        @pl.when(w0 + 2 < _NWIN)
        def _():
            fire(w0 + 2, ba0, bb0, sa0, sb0)

        drain(ba1, bb1, sa1, sb1)
        m_ = extract(w0 + 1, ba1, bb1, n, m_)

        @pl.when(w0 + 3 < _NWIN)
        def _():
            fire(w0 + 3, ba1, bb1, sa1, sb1)

        return m_

    m = lax.fori_loop(0, _NWIN // 2, wbody, 0)
    drain(ba0, bb0, sa0, sb0)
    m = extract(_NWIN - 1, ba0, bb0, n, m)

    def drain16(q, _):
        pltpu.make_async_copy(oa.at[pl.ds(0, 256)], dummy_v, soa).wait()
        pltpu.make_async_copy(ob.at[pl.ds(0, 256)], dummy_v, sob).wait()
        return 0

    def drain1(q, _):
        pltpu.make_async_copy(
            oa.at[pl.ds(0, 16)], dummy_v.at[pl.ds(0, 16)], soa).wait()
        pltpu.make_async_copy(
            ob.at[pl.ds(0, 16)], dummy_v.at[pl.ds(0, 16)], sob).wait()
        return 0

    lax.fori_loop(0, m // 16, drain16, 0)
    lax.fori_loop(0, m % 16, drain1, 0)


_ROWS = _B * _D // 128
_BR = 256


def _tc_combine(t_ref, a_ref, b_ref, o_ref):
    li = lax.broadcasted_iota(jnp.int32, (128, 8), 0)
    ji = lax.broadcasted_iota(jnp.int32, (128, 8), 1)
    s = (li // 16 == ji).astype(jnp.float32)
    lit = lax.broadcasted_iota(jnp.int32, (8, 128), 1)
    jit = lax.broadcasted_iota(jnp.int32, (8, 128), 0)
    st = (lit // 16 == jit).astype(jnp.float32)
    t = jnp.dot(t_ref[...], st, preferred_element_type=jnp.float32)
    n = (lax.broadcasted_iota(jnp.int32, (_BR, 128), 1) % 16 + 1).astype(
        jnp.float32)
    x = (2.0 * math.pi / _PERIOD) * (n * t)
    acc = jnp.cos(x) * a_ref[...] + jnp.sin(x) * b_ref[...]
    o_ref[...] = jnp.dot(acc, s, preferred_element_type=jnp.float32)


def kernel(t, id, a_table, b_table):
    idx = id.reshape(-1).astype(jnp.int32)
    ra, rb = _sc_gather(idx, a_table.T, b_table.T)
    out2 = pl.pallas_call(
        _tc_combine,
        grid=(_ROWS // _BR,),
        in_specs=[
            pl.BlockSpec((_BR, 8), lambda i: (i, 0)),
            pl.BlockSpec((_BR, 128), lambda i: (i, 0)),
            pl.BlockSpec((_BR, 128), lambda i: (i, 0)),
        ],
        out_specs=pl.BlockSpec((_BR, 8), lambda i: (i, 0)),
        out_shape=jax.ShapeDtypeStruct((_ROWS, 8), jnp.float32),
    )(t.reshape(_ROWS, 8), ra.reshape(_ROWS, 128), rb.reshape(_ROWS, 128))
    return out2.reshape(_B, 1)


# speedup vs baseline: 9.2934x; 1.1075x over previous
import functools
import math

import jax
import jax.numpy as jnp
from jax import lax
from jax.experimental import pallas as pl
from jax.experimental.pallas import tpu as pltpu
from jax.experimental.pallas import tpu_sc as plsc

_B = 16384
_D = 16
_PERIOD = 365.25
_NC, _NS = 2, 16
_NW = _NC * _NS
_NTILE = 7813
_RANGE = 245
_WT = 6
_NWIN = 41
_MAXC = _NTILE - _WT
_FLT = 784

_mesh = plsc.VectorSubcoreMesh(core_axis_name="c", subcore_axis_name="s")
_lane = lambda: lax.broadcasted_iota(jnp.int32, (16,), 0)


def _splat(v):
    return jax.lax.broadcast(v, (16,))


def _take(v, i):
    return lax.gather(
        v, i.reshape(16, 1),
        lax.GatherDimensionNumbers(
            offset_dims=(), collapsed_slice_dims=(0,), start_index_map=(0,)),
        slice_sizes=(1,),
        mode=lax.GatherScatterMode.PROMISE_IN_BOUNDS)


@functools.partial(
    pl.kernel,
    mesh=_mesh,
    out_type=[
        jax.ShapeDtypeStruct((_B * _D,), jnp.float32),
        jax.ShapeDtypeStruct((_B * _D,), jnp.float32),
    ],
    scratch_types=[
        pltpu.VMEM((_B,), jnp.int32),
        pltpu.VMEM((_FLT + 16,), jnp.int32),
        pltpu.VMEM((_FLT + 16,), jnp.int32),
        pltpu.VMEM((_D, _WT * 128), jnp.float32),
        pltpu.VMEM((_D, _WT * 128), jnp.float32),
        pltpu.VMEM((_D, _WT * 128), jnp.float32),
        pltpu.VMEM((_D, _WT * 128), jnp.float32),
        pltpu.VMEM((_D, _WT * 128), jnp.float32),
        pltpu.VMEM((_D, _WT * 128), jnp.float32),
        pltpu.VMEM(((_FLT + 16) * _D,), jnp.float32),
        pltpu.VMEM(((_FLT + 16) * _D,), jnp.float32),
        pltpu.VMEM((256,), jnp.float32),
        pltpu.SemaphoreType.DMA,
        pltpu.SemaphoreType.DMA,
        pltpu.SemaphoreType.DMA,
        pltpu.SemaphoreType.DMA,
        pltpu.SemaphoreType.DMA,
        pltpu.SemaphoreType.DMA,
        pltpu.SemaphoreType.DMA,
        pltpu.SemaphoreType.DMA,
    ],
    compiler_params=pltpu.CompilerParams(needs_layout_passes=False),
)
def _sc_gather(idx_hbm, a_hbm, b_hbm, oa, ob,
               ids_v, flt_v, wl_v, ba0, bb0, ba1, bb1, ba2, bb2,
               ca_v, cb_v, dummy_v,
               sa0, sb0, sa1, sb1, sa2, sb2, soa, sob):
    wid = lax.axis_index("s") * _NC + lax.axis_index("c")
    lo = wid * _RANGE
    lane = _lane()

    def filt(g_, off):
        v = ids_v[pl.ds(g_ * 16, 16)]
        roff = v - lo * 128
        m = (roff >= 0) & (roff < _RANGE * 128)
        packed = (roff << 14) | (g_ * 16 + lane)
        plsc.store_compressed(
            flt_v.at[pl.ds(jnp.minimum(off, _FLT), 16)], packed, mask=m)
        cnt = plsc.all_reduce_population_count(m)
        return off + cnt[0]

    def fire(w, ba, bb, sa, sb):
        cs = pl.multiple_of(jnp.minimum(lo + w * _WT, _MAXC) * 128, 128)
        pltpu.async_copy(a_hbm.at[:, pl.ds(cs, _WT * 128)], ba, sa)
        pltpu.async_copy(b_hbm.at[:, pl.ds(cs, _WT * 128)], bb, sb)

    def drain(ba, bb, sa, sb):
        pltpu.make_async_copy(a_hbm.at[:, pl.ds(0, _WT * 128)], ba, sa).wait()
        pltpu.make_async_copy(b_hbm.at[:, pl.ds(0, _WT * 128)], bb, sb).wait()

    def extract(w, ba, bb, n, m0):
        cs_t = jnp.minimum(lo + w * _WT, _MAXC)
        sbase = cs_t - lo

        def scan(g_, off):
            p = flt_v[pl.ds(g_ * 16, 16)]
            ro = p >> 14
            m = (ro >= sbase * 128) & (ro < (sbase + _WT) * 128) & (
                g_ * 16 + lane < n)
            plsc.store_compressed(wl_v.at[pl.ds(off, 16)], p, mask=m)
            cnt = plsc.all_reduce_population_count(m)
            return off + cnt[0]

        nw = lax.fori_loop(0, (n + 15) // 16, scan, 0)

        def one(k, m_):
            ch = wl_v[pl.ds((k // 16) * 16, 16)]
            e = _take(ch, _splat(k % 16))
            col = (e >> 14) - sbase * 128
            bpos = e & 16383
            va = plsc.load_gather(ba, [lane, col])
            vb = plsc.load_gather(bb, [lane, col])
            slot = m_ * 16
            dst = _splat(slot) + lane
            plsc.store_scatter(ca_v, [dst], va)
            plsc.store_scatter(cb_v, [dst], vb)
            off = pl.multiple_of(bpos[0] * 16, 16)
            pltpu.async_copy(ca_v.at[pl.ds(slot, 16)], oa.at[pl.ds(off, 16)], soa)
            pltpu.async_copy(cb_v.at[pl.ds(slot, 16)], ob.at[pl.ds(off, 16)], sob)
            return m_ + 1

        return lax.fori_loop(0, nw, one, m0)

    bufs = ((ba0, bb0, sa0, sb0), (ba1, bb1, sa1, sb1), (ba2, bb2, sa2, sb2))
    for j, (ba, bb, sa, sb) in enumerate(bufs):
        fire(j, ba, bb, sa, sb)

    pltpu.sync_copy(idx_hbm, ids_v)
    n = jnp.minimum(lax.fori_loop(0, _B // 16, filt, 0), _FLT)

    def wbody(k, m_):
        for j, (ba, bb, sa, sb) in enumerate(bufs):
            w = 3 * k + j
            drain(ba, bb, sa, sb)
            m_ = extract(w, ba, bb, n, m_)

            @pl.when(w + 3 < _NWIN)
            def _():
                fire(w + 3, ba, bb, sa, sb)

        return m_

    m = lax.fori_loop(0, _NWIN // 3, wbody, 0)
    drain(ba0, bb0, sa0, sb0)
    m = extract(_NWIN - 2, ba0, bb0, n, m)
    drain(ba1, bb1, sa1, sb1)
    m = extract(_NWIN - 1, ba1, bb1, n, m)

    def drain16(q, _):
        pltpu.make_async_copy(oa.at[pl.ds(0, 256)], dummy_v, soa).wait()
        pltpu.make_async_copy(ob.at[pl.ds(0, 256)], dummy_v, sob).wait()
        return 0

    def drain1(q, _):
        pltpu.make_async_copy(
            oa.at[pl.ds(0, 16)], dummy_v.at[pl.ds(0, 16)], soa).wait()
        pltpu.make_async_copy(
            ob.at[pl.ds(0, 16)], dummy_v.at[pl.ds(0, 16)], sob).wait()
        return 0

    lax.fori_loop(0, m // 16, drain16, 0)
    lax.fori_loop(0, m % 16, drain1, 0)


_ROWS = _B * _D // 128
_BR = 256


def _tc_combine(t_ref, a_ref, b_ref, o_ref):
    li = lax.broadcasted_iota(jnp.int32, (128, 8), 0)
    ji = lax.broadcasted_iota(jnp.int32, (128, 8), 1)
    s = (li // 16 == ji).astype(jnp.float32)
    lit = lax.broadcasted_iota(jnp.int32, (8, 128), 1)
    jit = lax.broadcasted_iota(jnp.int32, (8, 128), 0)
    st = (lit // 16 == jit).astype(jnp.float32)
    t = jnp.dot(t_ref[...], st, preferred_element_type=jnp.float32)
    n = (lax.broadcasted_iota(jnp.int32, (_BR, 128), 1) % 16 + 1).astype(
        jnp.float32)
    x = (2.0 * math.pi / _PERIOD) * (n * t)
    acc = jnp.cos(x) * a_ref[...] + jnp.sin(x) * b_ref[...]
    o_ref[...] = jnp.dot(acc, s, preferred_element_type=jnp.float32)


def kernel(t, id, a_table, b_table):
    idx = id.reshape(-1).astype(jnp.int32)
    ra, rb = _sc_gather(idx, a_table.T, b_table.T)
    out2 = pl.pallas_call(
        _tc_combine,
        grid=(_ROWS // _BR,),
        in_specs=[
            pl.BlockSpec((_BR, 8), lambda i: (i, 0)),
            pl.BlockSpec((_BR, 128), lambda i: (i, 0)),
            pl.BlockSpec((_BR, 128), lambda i: (i, 0)),
        ],
        out_specs=pl.BlockSpec((_BR, 8), lambda i: (i, 0)),
        out_shape=jax.ShapeDtypeStruct((_ROWS, 8), jnp.float32),
    )(t.reshape(_ROWS, 8), ra.reshape(_ROWS, 128), rb.reshape(_ROWS, 128))
    return out2.reshape(_B, 1)
